# baseline (speedup 1.0000x reference)
import functools

import jax
import jax.numpy as jnp
from jax.experimental import pallas as pl
from jax.experimental.pallas import tpu as pltpu

PAGE_SIZE = 16


def _append_body(num_pages, heads, hdim, T,
                 pages_ref, slots_ref, full_ref, nfull_ref, sing_ref,
                 nsing_ref, k_ref, v_ref, cache_hbm, out_hbm,
                 buf, in_sems, out_sems, patch_sem):
    PB = 32
    NBUF = 4
    nsteps = num_pages // PB

    def in_copy(i):
        return pltpu.make_async_copy(
            cache_hbm.at[pl.ds(i * PB, PB)], buf.at[i % NBUF],
            in_sems.at[i % NBUF])

    def out_copy(i):
        return pltpu.make_async_copy(
            buf.at[i % NBUF], out_hbm.at[pl.ds(i * PB, PB)],
            out_sems.at[i % NBUF])

    for i in range(min(3, nsteps)):
        in_copy(i).start()
    for i in range(nsteps):
        in_copy(i).wait()
        out_copy(i).start()
        if i + 3 < nsteps:
            if i >= 1:
                out_copy(i - 1).wait()
            in_copy(i + 3).start()
    for i in range(max(0, nsteps - 4), nsteps):
        out_copy(i).wait()

    def full_k(i, t0, p):
        return pltpu.make_async_copy(
            k_ref.at[pl.ds(t0, PAGE_SIZE)], out_hbm.at[p, 0], patch_sem)

    def full_v(i, t0, p):
        return pltpu.make_async_copy(
            v_ref.at[pl.ds(t0, PAGE_SIZE)], out_hbm.at[p, 1], patch_sem)

    def sing_k(t, p, s):
        return pltpu.make_async_copy(
            k_ref.at[pl.ds(t, 1)], out_hbm.at[p, 0, pl.ds(s, 1)], patch_sem)

    def sing_v(t, p, s):
        return pltpu.make_async_copy(
            v_ref.at[pl.ds(t, 1)], out_hbm.at[p, 1, pl.ds(s, 1)], patch_sem)

    max_full = T // PAGE_SIZE
    for phase in (0, 1):
        for i in range(max_full):
            @pl.when(i < nfull_ref[0])
            def _(i=i, phase=phase):
                t0 = full_ref[i]
                p = pages_ref[t0]
                if phase == 0:
                    full_k(i, t0, p).start()
                    full_v(i, t0, p).start()
                else:
                    full_k(i, t0, p).wait()
                    full_v(i, t0, p).wait()
        for j in range(T):
            @pl.when(j < nsing_ref[0])
            def _(j=j, phase=phase):
                t = sing_ref[j]
                p = pages_ref[t]
                s = slots_ref[t]
                if phase == 0:
                    sing_k(t, p, s).start()
                    sing_v(t, p, s).start()
                else:
                    sing_k(t, p, s).wait()
                    sing_v(t, p, s).wait()


def kernel(k, v, kv_cache, kv_append_indptr, kv_page_indices, kv_page_indptr,
           kv_page_lastlen):
    T, heads, hdim = k.shape
    num_pages = kv_cache.shape[0]

    tok = jnp.arange(T, dtype=jnp.int32)
    seq = jnp.searchsorted(kv_append_indptr, tok, side="right").astype(jnp.int32) - 1
    local = tok - kv_append_indptr[seq]
    n_new = kv_append_indptr[seq + 1] - kv_append_indptr[seq]
    n_pages = kv_page_indptr[seq + 1] - kv_page_indptr[seq]
    seq_total = (n_pages - 1) * PAGE_SIZE + kv_page_lastlen[seq]
    pos = seq_total - n_new + local
    page = kv_page_indices[kv_page_indptr[seq] + pos // PAGE_SIZE]
    slot = (pos % PAGE_SIZE).astype(jnp.int32)

    pg_pad = jnp.concatenate([page, jnp.full((PAGE_SIZE,), -1, jnp.int32)])
    sl_pad = jnp.concatenate([slot, jnp.full((PAGE_SIZE,), -1, jnp.int32)])
    full_start = (slot == 0) & (tok + PAGE_SIZE <= T)
    for j in range(1, PAGE_SIZE):
        full_start &= (pg_pad[tok + j] == page) & (sl_pad[tok + j] == j)
    fs_pad = jnp.concatenate([jnp.zeros((PAGE_SIZE - 1,), bool), full_start])
    covered = jnp.stack(
        [fs_pad[i:i + T] for i in range(PAGE_SIZE)], axis=0).any(axis=0)

    max_full = T // PAGE_SIZE
    full_idx = jnp.sort(jnp.where(full_start, tok, T))[:max_full]
    n_full = jnp.sum(full_start, dtype=jnp.int32).reshape(1)
    sing_idx = jnp.sort(jnp.where(~covered, tok, T))
    n_sing = jnp.sum(~covered, dtype=jnp.int32).reshape(1)

    smem = pl.BlockSpec(memory_space=pltpu.SMEM)
    body = functools.partial(_append_body, num_pages, heads, hdim, T)
    return pl.pallas_call(
        body,
        in_specs=[smem, smem, smem, smem, smem, smem,
                  pl.BlockSpec(memory_space=pltpu.MemorySpace.VMEM),
                  pl.BlockSpec(memory_space=pltpu.MemorySpace.VMEM),
                  pl.BlockSpec(memory_space=pl.ANY)],
        out_specs=pl.BlockSpec(memory_space=pl.ANY),
        out_shape=jax.ShapeDtypeStruct(kv_cache.shape, kv_cache.dtype),
        scratch_shapes=[
            pltpu.VMEM((4, 32, 2, PAGE_SIZE, heads, hdim), kv_cache.dtype),
            pltpu.SemaphoreType.DMA((4,)),
            pltpu.SemaphoreType.DMA((4,)),
            pltpu.SemaphoreType.DMA,
        ],
    )(page, slot, full_idx, n_full, sing_idx, n_sing, k, v, kv_cache)


# speedup vs baseline: 1.2845x; 1.2845x over previous
import functools

import jax
import jax.numpy as jnp
from jax.experimental import pallas as pl
from jax.experimental.pallas import tpu as pltpu

PAGE_SIZE = 16


def _copy_body(in_ref, out_ref):
    out_ref[...] = in_ref[...]


def _patch_body(T,
                pages_ref, slots_ref, full_ref, sing_ref,
                k_ref, v_ref, cache_hbm, out_hbm, patch_sem):
    def full_copy(kv_ref, t0, p, half):
        return pltpu.make_async_copy(
            kv_ref.at[pl.ds(t0, PAGE_SIZE)], out_hbm.at[p, half], patch_sem)

    def sing_copy(kv_ref, t, p, s, half):
        return pltpu.make_async_copy(
            kv_ref.at[pl.ds(t, 1)], out_hbm.at[p, half, pl.ds(s, 1)],
            patch_sem)

    def full_iter(phase):
        def body(t, carry):
            @pl.when(full_ref[t] == 1)
            def _():
                p = pages_ref[t]
                for half, ref in ((0, k_ref), (1, v_ref)):
                    c = full_copy(ref, t, p, half)
                    c.start() if phase == 0 else c.wait()
            return carry
        return body

    def sing_iter(phase):
        def body(t, carry):
            @pl.when(sing_ref[t] == 1)
            def _():
                p = pages_ref[t]
                s = slots_ref[t]
                for half, ref in ((0, k_ref), (1, v_ref)):
                    c = sing_copy(ref, t, p, s, half)
                    c.start() if phase == 0 else c.wait()
            return carry
        return body

    for phase in (0, 1):
        jax.lax.fori_loop(0, T, full_iter(phase), 0)
        jax.lax.fori_loop(0, T, sing_iter(phase), 0)


def kernel(k, v, kv_cache, kv_append_indptr, kv_page_indices, kv_page_indptr,
           kv_page_lastlen):
    T, heads, hdim = k.shape
    num_pages = kv_cache.shape[0]

    tok = jnp.arange(T, dtype=jnp.int32)
    seq = jnp.sum(tok[:, None] >= kv_append_indptr[None, 1:],
                  axis=1, dtype=jnp.int32)
    local = tok - kv_append_indptr[seq]
    n_new = kv_append_indptr[seq + 1] - kv_append_indptr[seq]
    n_pages = kv_page_indptr[seq + 1] - kv_page_indptr[seq]
    seq_total = (n_pages - 1) * PAGE_SIZE + kv_page_lastlen[seq]
    pos = seq_total - n_new + local
    page = kv_page_indices[kv_page_indptr[seq] + pos // PAGE_SIZE]
    slot = (pos % PAGE_SIZE).astype(jnp.int32)

    pg_pad = jnp.concatenate([page, jnp.full((PAGE_SIZE,), -1, jnp.int32)])
    sl_pad = jnp.concatenate([slot, jnp.full((PAGE_SIZE,), -1, jnp.int32)])
    full_start = (slot == 0) & (tok + PAGE_SIZE <= T)
    for j in range(1, PAGE_SIZE):
        full_start &= (pg_pad[j:j + T] == page) & (sl_pad[j:j + T] == j)
    fs_pad = jnp.concatenate([jnp.zeros((PAGE_SIZE - 1,), bool), full_start])
    covered = jnp.stack(
        [fs_pad[i:i + T] for i in range(PAGE_SIZE)], axis=0).any(axis=0)

    full_mask = full_start.astype(jnp.int32)
    sing_mask = (~covered).astype(jnp.int32)

    PB = 64
    copied = pl.pallas_call(
        _copy_body,
        grid=(num_pages // PB,),
        in_specs=[pl.BlockSpec((PB, 2, PAGE_SIZE, heads, hdim),
                               lambda i: (i, 0, 0, 0, 0))],
        out_specs=pl.BlockSpec((PB, 2, PAGE_SIZE, heads, hdim),
                               lambda i: (i, 0, 0, 0, 0)),
        out_shape=jax.ShapeDtypeStruct(kv_cache.shape, kv_cache.dtype),
        compiler_params=pltpu.CompilerParams(
            dimension_semantics=("arbitrary",)),
    )(kv_cache)

    smem = pl.BlockSpec(memory_space=pltpu.SMEM)
    return pl.pallas_call(
        functools.partial(_patch_body, T),
        in_specs=[smem, smem, smem, smem,
                  pl.BlockSpec(memory_space=pltpu.MemorySpace.VMEM),
                  pl.BlockSpec(memory_space=pltpu.MemorySpace.VMEM),
                  pl.BlockSpec(memory_space=pl.ANY)],
        out_specs=pl.BlockSpec(memory_space=pl.ANY),
        out_shape=jax.ShapeDtypeStruct(kv_cache.shape, kv_cache.dtype),
        input_output_aliases={6: 0},
        scratch_shapes=[pltpu.SemaphoreType.DMA],
    )(page, slot, full_mask, sing_mask, k, v, copied)


# speedup vs baseline: 1.5137x; 1.1785x over previous
import jax
import jax.numpy as jnp
from jax.experimental import pallas as pl
from jax.experimental.pallas import tpu as pltpu

PAGE_SIZE = 16


def _copy_body(in_ref, out_ref):
    out_ref[...] = in_ref[...]


def _patch_body(aip_ref, pgi_ref, pip_ref, ll_ref,
                k_ref, v_ref, cache_hbm, out_hbm, patch_sem):
    nseq = ll_ref.shape[0]

    def full_copy(kv_ref, t0, p, half):
        return pltpu.make_async_copy(
            kv_ref.at[pl.ds(t0, PAGE_SIZE)], out_hbm.at[p, half], patch_sem)

    def sing_copy(kv_ref, t, p, s, half):
        return pltpu.make_async_copy(
            kv_ref.at[pl.ds(t, 1)], out_hbm.at[p, half, pl.ds(s, 1)],
            patch_sem)

    def seq_iter(phase):
        def body(b, carry):
            start = aip_ref[b]
            end = aip_ref[b + 1]
            n_new = end - start
            n_pages = pip_ref[b + 1] - pip_ref[b]
            seq_total = (n_pages - 1) * PAGE_SIZE + ll_ref[b]
            base_pos = seq_total - n_new
            tbl = pip_ref[b]

            head = jnp.minimum(n_new, (-base_pos) % PAGE_SIZE)
            n_full = jnp.maximum(n_new - head, 0) // PAGE_SIZE

            def do_single(t, carry):
                pos = base_pos + (t - start)
                p = pgi_ref[tbl + pos // PAGE_SIZE]
                s = pos % PAGE_SIZE
                for half, ref in ((0, k_ref), (1, v_ref)):
                    c = sing_copy(ref, t, p, s, half)
                    c.start() if phase == 0 else c.wait()
                return carry

            def do_full(i, carry):
                t0 = start + head + i * PAGE_SIZE
                pos0 = base_pos + head + i * PAGE_SIZE
                p = pgi_ref[tbl + pos0 // PAGE_SIZE]
                for half, ref in ((0, k_ref), (1, v_ref)):
                    c = full_copy(ref, t0, p, half)
                    c.start() if phase == 0 else c.wait()
                return carry

            jax.lax.fori_loop(start, start + head, do_single, 0)
            jax.lax.fori_loop(0, n_full, do_full, 0)
            jax.lax.fori_loop(start + head + n_full * PAGE_SIZE, end,
                              do_single, 0)
            return carry
        return body

    for phase in (0, 1):
        jax.lax.fori_loop(0, nseq, seq_iter(phase), 0)


def kernel(k, v, kv_cache, kv_append_indptr, kv_page_indices, kv_page_indptr,
           kv_page_lastlen):
    T, heads, hdim = k.shape
    num_pages = kv_cache.shape[0]

    PB = 64
    copied = pl.pallas_call(
        _copy_body,
        grid=(num_pages // PB,),
        in_specs=[pl.BlockSpec((PB, 2, PAGE_SIZE, heads, hdim),
                               lambda i: (i, 0, 0, 0, 0))],
        out_specs=pl.BlockSpec((PB, 2, PAGE_SIZE, heads, hdim),
                               lambda i: (i, 0, 0, 0, 0)),
        out_shape=jax.ShapeDtypeStruct(kv_cache.shape, kv_cache.dtype),
        compiler_params=pltpu.CompilerParams(
            dimension_semantics=("arbitrary",)),
    )(kv_cache)

    smem = pl.BlockSpec(memory_space=pltpu.SMEM)
    return pl.pallas_call(
        _patch_body,
        in_specs=[smem, smem, smem, smem,
                  pl.BlockSpec(memory_space=pltpu.MemorySpace.VMEM),
                  pl.BlockSpec(memory_space=pltpu.MemorySpace.VMEM),
                  pl.BlockSpec(memory_space=pl.ANY)],
        out_specs=pl.BlockSpec(memory_space=pl.ANY),
        out_shape=jax.ShapeDtypeStruct(kv_cache.shape, kv_cache.dtype),
        input_output_aliases={6: 0},
        scratch_shapes=[pltpu.SemaphoreType.DMA],
    )(kv_append_indptr, kv_page_indices, kv_page_indptr, kv_page_lastlen,
      k, v, copied)
